# baseline (speedup 1.0000x reference)
import functools

import jax
import jax.numpy as jnp
from jax import lax
from jax.experimental import pallas as pl
from jax.experimental.pallas import tpu as pltpu
from jax.experimental.pallas import tpu_sc as plsc

BATCH = 4096
N_WORDS = 50
EMB_DIM = 64
VOCAB = 1000000
NUM_CORES = 2
NUM_SUBCORES = 16
NW = NUM_CORES * NUM_SUBCORES
CHUNK = 128

_mesh = plsc.VectorSubcoreMesh(core_axis_name="c", subcore_axis_name="s")


@functools.partial(
    pl.kernel,
    mesh=_mesh,
    out_type=jax.ShapeDtypeStruct((N_WORDS, EMB_DIM, BATCH), jnp.float32),
    scratch_types=[
        pltpu.VMEM((N_WORDS, CHUNK), jnp.int32),
        pltpu.VMEM((N_WORDS, CHUNK), jnp.int32),
        pltpu.VMEM((2, CHUNK, 2 * EMB_DIM), jnp.float32),
        pltpu.VMEM((2, EMB_DIM, CHUNK), jnp.float32),
        pltpu.SemaphoreType.DMA,
        pltpu.SemaphoreType.DMA,
        pltpu.SemaphoreType.DMA,
        pltpu.SemaphoreType.DMA,
    ],
    compiler_params=pltpu.CompilerParams(
        use_tc_tiling_on_sc=True, needs_layout_passes=False),
)
def _emb_lookup(idx2_hbm, poff_hbm, table_hbm, out_hbm,
                idx2_v, poff_v, rows2_v, tiles_v, gsem0, gsem1, wsem0, wsem1):
    c_ax = lax.axis_index("c")
    s_ax = lax.axis_index("s")
    wid = s_ax * NUM_CORES + c_ax
    pltpu.sync_copy(idx2_hbm.at[wid], idx2_v)
    pltpu.sync_copy(poff_hbm.at[wid], poff_v)
    gsems = (gsem0, gsem1)
    wsems = (wsem0, wsem1)

    def g_desc(n, slot):
        return pltpu.make_async_copy(
            table_hbm.at[idx2_v.at[n]], rows2_v.at[slot], gsems[slot])

    def w_desc(n, slot):
        return pltpu.make_async_copy(
            tiles_v.at[slot],
            out_hbm.at[n, :, pl.ds(wid * CHUNK, CHUNK)],
            wsems[slot])

    lanes = lax.iota(jnp.int32, 16)

    def extract(n, slot):
        for g in range(CHUNK // 16):
            rvec = 16 * g + lanes
            cbase = poff_v[n, pl.ds(16 * g, 16)]
            for f in range(EMB_DIM):
                val = plsc.load_gather(rows2_v.at[slot], [rvec, cbase + f])
                tiles_v[slot, f, pl.ds(16 * g, 16)] = val

    def step(n, slot):
        @pl.when(n + 1 < N_WORDS)
        def _():
            g_desc(n + 1, 1 - slot).start()
        g_desc(n, slot).wait()

        @pl.when(n >= 2)
        def _():
            w_desc(n - 2, slot).wait()
        extract(n, slot)
        w_desc(n, slot).start()

    g_desc(0, 0).start()

    def body(k, carry):
        step(2 * k, 0)
        step(2 * k + 1, 1)
        return carry

    lax.fori_loop(0, N_WORDS // 2, body, 0)
    w_desc(N_WORDS - 2, 0).wait()
    w_desc(N_WORDS - 1, 1).wait()


def kernel(input_words, embeddings):
    iw = input_words.astype(jnp.int32).reshape(NW, CHUNK, N_WORDS)
    iw = iw.transpose(0, 2, 1)
    idx2 = iw >> 1
    poff = (iw & 1) << 6
    table2 = embeddings.reshape(VOCAB // 2, 2 * EMB_DIM)
    out = _emb_lookup(idx2, poff, table2)
    return out.transpose(2, 0, 1)


# speedup vs baseline: 1.2209x; 1.2209x over previous
import functools

import jax
import jax.numpy as jnp
from jax import lax
from jax.experimental import pallas as pl
from jax.experimental.pallas import tpu as pltpu
from jax.experimental.pallas import tpu_sc as plsc

BATCH = 4096
N_WORDS = 50
EMB_DIM = 64
VOCAB = 1000000
NUM_CORES = 2
NUM_SUBCORES = 16
NW = NUM_CORES * NUM_SUBCORES
CHUNK = 128

_mesh = plsc.VectorSubcoreMesh(core_axis_name="c", subcore_axis_name="s")


@functools.partial(
    pl.kernel,
    mesh=_mesh,
    out_type=jax.ShapeDtypeStruct((N_WORDS, EMB_DIM, BATCH), jnp.float32),
    scratch_types=[
        pltpu.VMEM((N_WORDS, CHUNK), jnp.int32),
        pltpu.VMEM((N_WORDS, CHUNK), jnp.int32),
        pltpu.VMEM((2, CHUNK, 2 * EMB_DIM), jnp.float32),
        pltpu.VMEM((2, EMB_DIM, CHUNK), jnp.float32),
        pltpu.SemaphoreType.DMA,
        pltpu.SemaphoreType.DMA,
        pltpu.SemaphoreType.DMA,
        pltpu.SemaphoreType.DMA,
    ],
    compiler_params=pltpu.CompilerParams(
        use_tc_tiling_on_sc=True, needs_layout_passes=False),
)
def _emb_lookup(idx2_hbm, poff_hbm, table_hbm, out_hbm,
                idx2_v, poff_v, rows2_v, tiles_v, gsem0, gsem1, wsem0, wsem1):
    c_ax = lax.axis_index("c")
    s_ax = lax.axis_index("s")
    wid = s_ax * NUM_CORES + c_ax
    pltpu.sync_copy(idx2_hbm.at[wid], idx2_v)
    pltpu.sync_copy(poff_hbm.at[wid], poff_v)
    gsems = (gsem0, gsem1)
    wsems = (wsem0, wsem1)

    def g_desc(n, slot):
        return pltpu.make_async_copy(
            table_hbm.at[idx2_v.at[n]], rows2_v.at[slot], gsems[slot])

    def w_desc(n, slot):
        return pltpu.make_async_copy(
            tiles_v.at[slot],
            out_hbm.at[n, :, pl.ds(wid * CHUNK, CHUNK)],
            wsems[slot])

    lanes = lax.iota(jnp.int32, 16)

    def extract(n, slot):
        rvecs = [16 * g + lanes for g in range(CHUNK // 16)]
        cbases = [poff_v[n, pl.ds(16 * g, 16)] for g in range(CHUNK // 16)]

        @plsc.parallel_loop(0, EMB_DIM, unroll=8)
        def _(f):
            for g in range(CHUNK // 16):
                val = plsc.load_gather(rows2_v.at[slot], [rvecs[g], cbases[g] + f])
                tiles_v[slot, f, pl.ds(16 * g, 16)] = val

    def step(n, slot):
        @pl.when(n + 1 < N_WORDS)
        def _():
            g_desc(n + 1, 1 - slot).start()
        g_desc(n, slot).wait()

        @pl.when(n >= 2)
        def _():
            w_desc(n - 2, slot).wait()
        extract(n, slot)
        w_desc(n, slot).start()

    g_desc(0, 0).start()

    def body(k, carry):
        step(2 * k, 0)
        step(2 * k + 1, 1)
        return carry

    lax.fori_loop(0, N_WORDS // 2, body, 0)
    w_desc(N_WORDS - 2, 0).wait()
    w_desc(N_WORDS - 1, 1).wait()


def kernel(input_words, embeddings):
    iw = input_words.astype(jnp.int32).reshape(NW, CHUNK, N_WORDS)
    iw = iw.transpose(0, 2, 1)
    idx2 = iw >> 1
    poff = (iw & 1) << 6
    table2 = embeddings.reshape(VOCAB // 2, 2 * EMB_DIM)
    out = _emb_lookup(idx2, poff, table2)
    return out.transpose(2, 0, 1)
